# initial kernel scaffold (unmeasured)
import jax
import jax.numpy as jnp
from jax import lax
from jax.experimental import pallas as pl
from jax.experimental.pallas import tpu as pltpu

N_DEV = 32


def kernel(x, w_mat):
    m_per, k = x.shape
    n = w_mat.shape[1]
    blk = n // N_DEV

    x_bf = x.astype(jnp.bfloat16)

    def body(x_ref, w_ref, out_ref, wbuf, ysend, comm, load_sems, copy_sem,
             send_sems, recv_sems):
        me = lax.axis_index("i")

        def w_load(s, slot):
            j = (me + s + 1) % N_DEV
            return pltpu.make_async_copy(
                w_ref.at[:, pl.ds(j * blk, blk)],
                wbuf.at[slot],
                load_sems.at[slot],
            )

        def send_desc(slot, peer):
            return pltpu.make_async_remote_copy(
                src_ref=ysend.at[slot],
                dst_ref=comm.at[me],
                send_sem=send_sems.at[slot],
                recv_sem=recv_sems.at[me],
                device_id=(peer,),
                device_id_type=pl.DeviceIdType.MESH,
            )

        w_load(0, 0).start()

        for s in range(N_DEV):
            slot = s % 2
            if s + 1 < N_DEV:
                w_load(s + 1, 1 - slot).start()
            w_load(s, slot).wait()
            wb = wbuf[slot].astype(jnp.bfloat16)
            yb = jnp.dot(x_ref[...], wb, preferred_element_type=jnp.float32)
            if s >= 2:
                send_desc(slot, me).wait_send()
            ysend[slot] = yb.astype(jnp.bfloat16)
            j = (me + s + 1) % N_DEV
            if s < N_DEV - 1:
                send_desc(slot, j).start()
            else:
                cp = pltpu.make_async_copy(ysend.at[slot], comm.at[me],
                                           copy_sem)
                cp.start()
                cp.wait()

        send_desc((N_DEV - 2) % 2, me).wait_send()

        for d in range(1, N_DEV):
            src = (me - d) % N_DEV
            pltpu.make_async_remote_copy(
                src_ref=ysend.at[0],
                dst_ref=comm.at[src],
                send_sem=send_sems.at[0],
                recv_sem=recv_sems.at[src],
                device_id=(src,),
                device_id_type=pl.DeviceIdType.MESH,
            ).wait_recv()

        v = comm[...].reshape(N_DEV * m_per, blk).astype(jnp.float32)
        out_ref[...] = v / (1.0 + jnp.exp(-v))

    return pl.pallas_call(
        body,
        out_shape=jax.ShapeDtypeStruct((N_DEV * m_per, blk), jnp.float32),
        in_specs=[
            pl.BlockSpec(memory_space=pltpu.VMEM),
            pl.BlockSpec(memory_space=pltpu.ANY),
        ],
        out_specs=pl.BlockSpec(memory_space=pltpu.VMEM),
        scratch_shapes=[
            pltpu.VMEM((2, k, blk), jnp.float32),
            pltpu.VMEM((2, m_per, blk), jnp.bfloat16),
            pltpu.VMEM((N_DEV, m_per, blk), jnp.bfloat16),
            pltpu.SemaphoreType.DMA((2,)),
            pltpu.SemaphoreType.DMA,
            pltpu.SemaphoreType.DMA((2,)),
            pltpu.SemaphoreType.DMA((N_DEV,)),
        ],
    )(x_bf, w_mat)


# baseline (device time: 67242 ns/iter reference)
import jax
import jax.numpy as jnp
from jax import lax
from jax.experimental import pallas as pl
from jax.experimental.pallas import tpu as pltpu

N_DEV = 32


def kernel(x, w_mat):
    m_per, k = x.shape
    n = w_mat.shape[1]
    blk = n // N_DEV

    x_bf = x.astype(jnp.bfloat16)

    def body(x_ref, w_ref, out_ref, wbuf, ysend, comm, load_sems, copy_sem,
             send_sems, recv_sems):
        me = lax.axis_index("i")

        def w_load(s, slot):
            j = (me + s + 1) % N_DEV
            return pltpu.make_async_copy(
                w_ref.at[:, pl.ds(j * blk, blk)],
                wbuf.at[slot],
                load_sems.at[slot],
            )

        def send_desc(slot, peer):
            return pltpu.make_async_remote_copy(
                src_ref=ysend.at[slot],
                dst_ref=comm.at[me],
                send_sem=send_sems.at[slot],
                recv_sem=recv_sems.at[me],
                device_id=(peer,),
                device_id_type=pl.DeviceIdType.MESH,
            )

        w_load(0, 0).start()

        for s in range(N_DEV):
            slot = s % 2
            if s + 1 < N_DEV:
                w_load(s + 1, 1 - slot).start()
            w_load(s, slot).wait()
            wb = wbuf[slot].astype(jnp.bfloat16)
            yb = jnp.dot(x_ref[...], wb, preferred_element_type=jnp.float32)
            if s >= 2:
                send_desc(slot, me).wait_send()
            ysend[slot] = yb.astype(jnp.bfloat16)
            j = (me + s + 1) % N_DEV
            if s < N_DEV - 1:
                send_desc(slot, j).start()
            else:
                cp = pltpu.make_async_copy(ysend.at[slot], comm.at[me],
                                           copy_sem)
                cp.start()
                cp.wait()

        send_desc((N_DEV - 2) % 2, me).wait_send()

        for d in range(1, N_DEV):
            src = (me - d) % N_DEV
            pltpu.make_async_remote_copy(
                src_ref=ysend.at[0],
                dst_ref=comm.at[src],
                send_sem=send_sems.at[0],
                recv_sem=recv_sems.at[src],
                device_id=(src,),
                device_id_type=pl.DeviceIdType.MESH,
            ).wait_recv()

        v = comm[...].reshape(N_DEV * m_per, blk).astype(jnp.float32)
        out_ref[...] = v / (1.0 + jnp.exp(-v))

    return pl.pallas_call(
        body,
        out_shape=jax.ShapeDtypeStruct((N_DEV * m_per, blk), jnp.float32),
        in_specs=[
            pl.BlockSpec(memory_space=pltpu.VMEM),
            pl.BlockSpec(memory_space=pl.ANY),
        ],
        out_specs=pl.BlockSpec(memory_space=pltpu.VMEM),
        scratch_shapes=[
            pltpu.VMEM((2, k, blk), jnp.float32),
            pltpu.VMEM((2, m_per, blk), jnp.bfloat16),
            pltpu.VMEM((N_DEV, m_per, blk), jnp.bfloat16),
            pltpu.SemaphoreType.DMA((2,)),
            pltpu.SemaphoreType.DMA,
            pltpu.SemaphoreType.DMA((2,)),
            pltpu.SemaphoreType.DMA((N_DEV,)),
        ],
    )(x_bf, w_mat)
